# baseline (speedup 1.0000x reference)
import functools

import jax
import jax.numpy as jnp
from jax import lax
from jax.experimental import pallas as pl
from jax.experimental.pallas import tpu as pltpu
from jax.experimental.pallas import tpu_sc as plsc

NEG_SLOPE = 0.01
BN_EPS = 1e-5

F_CHUNK = 128
EDGE_BLK = 80
NUM_SC = 2
NUM_TILES = 16
ROW_ALIGN = 8 * NUM_TILES


def _lrelu(x):
    return jnp.where(x >= 0, x, NEG_SLOPE * x)


def _bn_cols(x, g, b):
    mu = jnp.mean(x, axis=0, keepdims=True)
    xc = x - mu
    var = jnp.mean(xc * xc, axis=0, keepdims=True)
    return xc * lax.rsqrt(var + BN_EPS) * g + b


def _pre_body(h_ref, g_ref, b_ref, *out_refs, nc, first):
    hn = _bn_cols(h_ref[...], g_ref[...], b_ref[...])
    if not first:
        hn = _lrelu(hn)
    n = h_ref.shape[0]
    for ci in range(nc):
        out_refs[ci][:n, :] = hn[:, ci * F_CHUNK:(ci + 1) * F_CHUNK]


def _pre(h, g, b, first):
    n, in_ch = h.shape
    n_pad = ((n + ROW_ALIGN - 1) // ROW_ALIGN) * ROW_ALIGN
    nc = in_ch // F_CHUNK
    return pl.pallas_call(
        functools.partial(_pre_body, nc=nc, first=first),
        out_shape=[jax.ShapeDtypeStruct((n_pad, F_CHUNK), jnp.float32)
                   for _ in range(nc)],
    )(h, g.reshape(1, -1), b.reshape(1, -1))


def _elin_body(ea_ref, w_ref, b_ref, *out_refs, nc):
    e = jnp.dot(ea_ref[...], w_ref[...], preferred_element_type=jnp.float32)
    e = e + b_ref[...]
    for ci in range(nc):
        out_refs[ci][...] = e[:, ci * F_CHUNK:(ci + 1) * F_CHUNK]


def _elin(edge_attr, w, b):
    e_total, d_edge = edge_attr.shape
    in_ch = w.shape[1]
    nc = in_ch // F_CHUNK
    eb = 4000 if e_total % 4000 == 0 else e_total
    grid = e_total // eb
    return pl.pallas_call(
        functools.partial(_elin_body, nc=nc),
        grid=(grid,),
        in_specs=[
            pl.BlockSpec((eb, d_edge), lambda i: (i, 0)),
            pl.BlockSpec((d_edge, in_ch), lambda i: (0, 0)),
            pl.BlockSpec((1, in_ch), lambda i: (0, 0)),
        ],
        out_specs=[pl.BlockSpec((eb, F_CHUNK), lambda i: (i, 0))
                   for _ in range(nc)],
        out_shape=[jax.ShapeDtypeStruct((e_total, F_CHUNK), jnp.float32)
                   for _ in range(nc)],
    )(edge_attr, w, b.reshape(1, -1))


def _sc_edge_body(*refs, nc, n_nodes, n_edges):
    hin = refs[:nc]
    e = refs[nc:2 * nc]
    idx_h, zeros_h = refs[2 * nc:2 * nc + 2]
    outs = refs[2 * nc + 2:3 * nc + 2]
    sc = refs[3 * nc + 2:]
    idxb = sc[0:3]
    rows = sc[3:6]
    a_sp = sc[6]
    isem = sc[7:10]
    esem = sc[10:13]
    gsem = sc[13:16]
    ssem = sc[16:19]

    c = lax.axis_index("c")
    s = lax.axis_index("s")
    wid = c * NUM_TILES + s
    ept = n_edges // (NUM_SC * NUM_TILES)
    nblk = ept // EDGE_BLK
    rpt = n_nodes // NUM_TILES
    base = wid * ept

    for ci in range(nc):
        pltpu.sync_copy(zeros_h.at[pl.ds(s * rpt, rpt), :],
                        a_sp.at[pl.ds(s * rpt, rpt), :])
        plsc.subcore_barrier()

        def e_slice(j, ci=ci):
            return e[ci].at[pl.ds(base + j * EDGE_BLK, EDGE_BLK), :]

        def issue_feed(j, r):
            pltpu.async_copy(idx_h.at[wid, j], idxb[r], isem[r])
            pltpu.async_copy(e_slice(j), rows[r], esem[r])

        def issue_gather(j, r, ci=ci):
            pltpu.async_copy(hin[ci].at[idxb[r].at[0]], rows[r], gsem[r],
                             add=True)

        def wait_feed(j, r):
            pltpu.make_async_copy(idx_h.at[wid, j], idxb[r], isem[r]).wait()
            pltpu.make_async_copy(e_slice(j), rows[r], esem[r]).wait()

        issue_feed(0, 0)
        issue_feed(1, 1)
        wait_feed(0, 0)
        issue_gather(0, 0)

        def group(g, _, ci=ci):
            for r in range(3):
                i = 3 * g + r
                rn = (r + 1) % 3
                rj = (r + 2) % 3

                @pl.when(i < nblk)
                def _(i=i, r=r, rn=rn, rj=rj, ci=ci):
                    @pl.when(i + 1 < nblk)
                    def _():
                        wait_feed(i + 1, rn)
                        issue_gather(i + 1, rn)

                    pltpu.make_async_copy(hin[ci].at[idxb[r].at[0]],
                                          rows[r], gsem[r]).wait()

                    @plsc.parallel_loop(0, EDGE_BLK, step=1, unroll=4)
                    def _(rr, r=r):
                        for cc in range(F_CHUNK // 16):
                            sl = pl.ds(cc * 16, 16)
                            rows[r][rr, sl] = jnp.maximum(rows[r][rr, sl], 0.0)
                    pltpu.async_copy(rows[r], a_sp.at[idxb[r].at[1]],
                                     ssem[r], add=True)

                    @pl.when(i + 2 < nblk)
                    def _():
                        @pl.when(i >= 1)
                        def _():
                            pltpu.make_async_copy(
                                rows[rj], a_sp.at[idxb[rj].at[1]],
                                ssem[rj]).wait()
                        issue_feed(i + 2, rj)
            return 0

        lax.fori_loop(0, (nblk + 2) // 3, group, 0)
        for r in range(3):
            pltpu.make_async_copy(rows[r], a_sp.at[idxb[r].at[1]],
                                  ssem[r]).wait()
        plsc.subcore_barrier()
        pltpu.sync_copy(a_sp.at[pl.ds(s * rpt, rpt), :],
                        outs[ci].at[c, pl.ds(s * rpt, rpt), :])
        if ci + 1 < nc:
            plsc.subcore_barrier()


def _sc_edge(hin_list, e_list, idx2, zeros):
    nc = len(hin_list)
    n_nodes = hin_list[0].shape[0]
    nw = NUM_SC * NUM_TILES
    nblk = idx2.shape[1]
    n_edges = nw * nblk * EDGE_BLK
    mesh = plsc.VectorSubcoreMesh(core_axis_name="c", subcore_axis_name="s",
                                  num_cores=NUM_SC, num_subcores=NUM_TILES)
    body = functools.partial(_sc_edge_body, nc=nc, n_nodes=n_nodes,
                             n_edges=n_edges)
    fn = pl.kernel(
        body,
        out_type=[jax.ShapeDtypeStruct((NUM_SC, n_nodes, F_CHUNK), jnp.float32)
                  for _ in range(nc)],
        mesh=mesh,
        scratch_types=(
            [pltpu.VMEM((2, EDGE_BLK), jnp.int32) for _ in range(3)]
            + [pltpu.VMEM((EDGE_BLK, F_CHUNK), jnp.float32) for _ in range(3)]
            + [pltpu.VMEM_SHARED((n_nodes, F_CHUNK), jnp.float32)]
            + [pltpu.SemaphoreType.DMA for _ in range(12)]
        ),
    )
    outs = fn(*hin_list, *e_list, idx2, zeros)
    return outs if isinstance(outs, (list, tuple)) else (outs,)


def _aggw1_body(*refs, nc):
    hin = refs[:nc]
    parts = refs[nc:2 * nc]
    w1_ref, b1_ref, eps_ref, out_ref = refs[2 * nc:]
    scale = 1.0 + eps_ref[0, 0]
    zs = [scale * hin[ci][...] + parts[ci][0] + parts[ci][1]
          for ci in range(nc)]
    z = jnp.concatenate(zs, axis=1)
    out_ref[...] = (jnp.dot(z, w1_ref[...], preferred_element_type=jnp.float32)
                    + b1_ref[...])


def _agg_w1(hin_list, parts, w1, b1, eps, n):
    nc = len(hin_list)
    in_ch, h_dim = w1.shape
    rb = 2000
    grid = n // rb
    return pl.pallas_call(
        functools.partial(_aggw1_body, nc=nc),
        grid=(grid,),
        in_specs=(
            [pl.BlockSpec((rb, F_CHUNK), lambda i: (i, 0))] * nc
            + [pl.BlockSpec((NUM_SC, rb, F_CHUNK), lambda i: (0, i, 0))] * nc
            + [pl.BlockSpec((in_ch, h_dim), lambda i: (0, 0)),
               pl.BlockSpec((1, h_dim), lambda i: (0, 0)),
               pl.BlockSpec((1, 1), lambda i: (0, 0))]
        ),
        out_specs=pl.BlockSpec((rb, h_dim), lambda i: (i, 0)),
        out_shape=jax.ShapeDtypeStruct((n, h_dim), jnp.float32),
    )(*hin_list, *parts, w1, b1.reshape(1, -1), eps.reshape(1, 1))


def _post_body(z1_ref, g_ref, b_ref, w2_ref, b2_ref, hprev_ref, out_ref):
    zn = _lrelu(_bn_cols(z1_ref[...], g_ref[...], b_ref[...]))
    z2 = jnp.dot(zn, w2_ref[...], preferred_element_type=jnp.float32) + b2_ref[...]
    in_ch = hprev_ref.shape[1]
    out_ref[:, :in_ch] = hprev_ref[...]
    out_ref[:, in_ch:] = z2


def _post(z1, g, b, w2, b2, hprev):
    n, in_ch = hprev.shape
    h_dim = w2.shape[1]
    return pl.pallas_call(
        _post_body,
        out_shape=jax.ShapeDtypeStruct((n, in_ch + h_dim), jnp.float32),
    )(z1, g.reshape(1, -1), b.reshape(1, -1), w2, b2.reshape(1, -1), hprev)


def _clf_body(h_ref, w1_ref, b1_ref, g1_ref, be1_ref, w2_ref, b2_ref,
              g2_ref, be2_ref, w3_ref, b3_ref, out_ref):
    y = jnp.dot(h_ref[...], w1_ref[...], preferred_element_type=jnp.float32)
    y = _lrelu(_bn_cols(y + b1_ref[...], g1_ref[...], be1_ref[...]))
    y = jnp.dot(y, w2_ref[...], preferred_element_type=jnp.float32)
    y = _lrelu(_bn_cols(y + b2_ref[...], g2_ref[...], be2_ref[...]))
    out_ref[...] = (jnp.dot(y, w3_ref[...], preferred_element_type=jnp.float32)
                    + b3_ref[...])


def _clf(h, c):
    n = h.shape[0]
    return pl.pallas_call(
        _clf_body,
        out_shape=jax.ShapeDtypeStruct((n, 1), jnp.float32),
    )(h, c['W1'], c['b1'].reshape(1, -1), c['g1'].reshape(1, -1),
      c['be1'].reshape(1, -1), c['W2'], c['b2'].reshape(1, -1),
      c['g2'].reshape(1, -1), c['be2'].reshape(1, -1), c['W3'],
      c['b3'].reshape(1, -1))


def kernel(x, edge_index, edge_attr, params):
    src = edge_index[0]
    dst = edge_index[1]
    n = x.shape[0]
    n_pad = ((n + ROW_ALIGN - 1) // ROW_ALIGN) * ROW_ALIGN
    zeros = jnp.zeros((n_pad, F_CHUNK), jnp.float32)
    nw = NUM_SC * NUM_TILES
    nblk = src.shape[0] // (nw * EDGE_BLK)
    idx2 = jnp.stack([src.reshape(nw, nblk, EDGE_BLK),
                      dst.reshape(nw, nblk, EDGE_BLK)], axis=2)
    h = x
    num_layers = sum(1 for k in params if k.startswith('layer'))
    e_all = [_elin(edge_attr, params['layer%d' % i]['eW'],
                   params['layer%d' % i]['eb']) for i in range(num_layers)]
    for i in range(num_layers):
        p = params['layer%d' % i]
        hin_c = _pre(h, p['bn_g'], p['bn_b'], first=(i == 0))
        parts = _sc_edge(hin_c, e_all[i], idx2, zeros)
        z1 = _agg_w1(hin_c, parts, p['W1'], p['b1'], p['eps'], n)
        h = _post(z1, p['nbn_g'], p['nbn_b'], p['W2'], p['b2'], h)
    y = _clf(h, params['clf'])
    return y.reshape(-1)
